# baseline (speedup 1.0000x reference)
import functools

import jax
import jax.numpy as jnp
from jax import lax
from jax.experimental import pallas as pl
from jax.experimental.pallas import tpu as pltpu
from jax.experimental.pallas import tpu_sc as plsc

N = 10000
E = 320000
D = 128

NC = 2
NS = 16
NW = NC * NS
K = 128
NBLK = 80
E_PAD = NW * NBLK * K
N_PAD = 10240
R_SUB = N_PAD // NS
NCHUNK = 2
BPC = NBLK // NCHUNK

_mesh = plsc.VectorSubcoreMesh(
    core_axis_name="c", subcore_axis_name="s", num_cores=NC, num_subcores=NS
)


@functools.partial(
    pl.kernel,
    out_type=jax.ShapeDtypeStruct((NC, N_PAD), jnp.float32),
    mesh=_mesh,
    scratch_types=[
        pltpu.VMEM((NBLK, K), jnp.int32),
        pltpu.VMEM((R_SUB,), jnp.float32),
        pltpu.VMEM((K,), jnp.float32),
        pltpu.VMEM_SHARED((N_PAD,), jnp.float32),
    ],
)
def _deg_kernel(dst_hbm, out_hbm, dst_v, zero_v, ones_v, acc):
    cid = lax.axis_index("c")
    sid = lax.axis_index("s")
    wid = sid * NC + cid

    def fill(i, _):
        zero_v[pl.ds(i * 16, 16)] = jnp.zeros((16,), jnp.float32)
        return 0

    lax.fori_loop(0, R_SUB // 16, fill, 0)

    def fill1(i, _):
        ones_v[pl.ds(i * 16, 16)] = jnp.ones((16,), jnp.float32)
        return 0

    lax.fori_loop(0, K // 16, fill1, 0)

    pltpu.sync_copy(dst_hbm.at[wid], dst_v)
    pltpu.sync_copy(zero_v, acc.at[pl.ds(sid * R_SUB, R_SUB)])
    plsc.subcore_barrier()

    def body(j, _):
        pltpu.sync_copy(ones_v, acc.at[dst_v.at[j]], add=True)
        return 0

    lax.fori_loop(0, NBLK, body, 0)
    plsc.subcore_barrier()
    pltpu.sync_copy(
        acc.at[pl.ds(sid * R_SUB, R_SUB)],
        out_hbm.at[cid, pl.ds(sid * R_SUB, R_SUB)],
    )


@functools.partial(
    pl.kernel,
    out_type=jax.ShapeDtypeStruct((NC, N_PAD, D), jnp.float32),
    mesh=_mesh,
    scratch_types=[
        pltpu.VMEM((BPC, K), jnp.int32),
        pltpu.VMEM((BPC, K), jnp.int32),
        pltpu.VMEM((K, D), jnp.float32),
        pltpu.VMEM((K, D), jnp.float32),
        pltpu.VMEM_SHARED((N_PAD, D), jnp.float32),
        pltpu.SemaphoreType.DMA,
        pltpu.SemaphoreType.DMA,
    ],
)
def _agg_kernel(hp_hbm, src_hbm, dst_hbm, out_hbm,
                src_v, dst_v, buf0, buf1, acc, sem0, sem1):
    cid = lax.axis_index("c")
    sid = lax.axis_index("s")
    wid = sid * NC + cid

    def fill(i, _):
        buf0[i // 8, pl.ds((i % 8) * 16, 16)] = jnp.zeros((16,), jnp.float32)
        return 0

    lax.fori_loop(0, K * 8, fill, 0)
    for t in range(R_SUB // K):
        pltpu.sync_copy(buf0, acc.at[pl.ds(sid * R_SUB + t * K, K)])
    plsc.subcore_barrier()

    def chunk(c, _):
        pltpu.sync_copy(src_hbm.at[wid, pl.ds(c * BPC, BPC)], src_v)
        pltpu.sync_copy(dst_hbm.at[wid, pl.ds(c * BPC, BPC)], dst_v)
        pltpu.async_copy(hp_hbm.at[src_v.at[0]], buf0, sem0)
        pltpu.async_copy(hp_hbm.at[src_v.at[1]], buf1, sem1)

        def body(g, _):
            a = 2 * g
            pltpu.make_async_copy(hp_hbm.at[src_v.at[a]], buf0, sem0).wait()
            pltpu.async_copy(buf0, acc.at[dst_v.at[a]], sem0, add=True)
            pltpu.make_async_copy(hp_hbm.at[src_v.at[a + 1]], buf1, sem1).wait()
            pltpu.async_copy(buf1, acc.at[dst_v.at[a + 1]], sem1, add=True)

            @pl.when(g < BPC // 2 - 1)
            def _():
                pltpu.make_async_copy(buf0, acc.at[dst_v.at[a]], sem0).wait()
                pltpu.async_copy(hp_hbm.at[src_v.at[a + 2]], buf0, sem0)
                pltpu.make_async_copy(buf1, acc.at[dst_v.at[a + 1]], sem1).wait()
                pltpu.async_copy(hp_hbm.at[src_v.at[a + 3]], buf1, sem1)

            return 0

        lax.fori_loop(0, BPC // 2, body, 0)
        pltpu.make_async_copy(buf0, acc.at[dst_v.at[BPC - 2]], sem0).wait()
        pltpu.make_async_copy(buf1, acc.at[dst_v.at[BPC - 1]], sem1).wait()
        return 0

    lax.fori_loop(0, NCHUNK, chunk, 0)
    plsc.subcore_barrier()
    pltpu.sync_copy(
        acc.at[pl.ds(sid * R_SUB, R_SUB)],
        out_hbm.at[cid, pl.ds(sid * R_SUB, R_SUB)],
    )


_GB = 10
_RB = N_PAD // _GB


def _tc1_body(deg_ref, x_ref, w_ref, hp_ref, dis_ref):
    deg = deg_ref[0] + deg_ref[1] + 1.0
    dis = lax.rsqrt(deg)
    dis_ref[...] = dis
    hp_ref[...] = jnp.dot(
        x_ref[...], w_ref[...], preferred_element_type=jnp.float32
    ) * dis


def _tc1(deg3, x_p, W1):
    return pl.pallas_call(
        _tc1_body,
        grid=(_GB,),
        in_specs=[
            pl.BlockSpec((NC, _RB, 1), lambda i: (0, i, 0)),
            pl.BlockSpec((_RB, D), lambda i: (i, 0)),
            pl.BlockSpec((D, D), lambda i: (0, 0)),
        ],
        out_specs=[
            pl.BlockSpec((_RB, D), lambda i: (i, 0)),
            pl.BlockSpec((_RB, 1), lambda i: (i, 0)),
        ],
        out_shape=[
            jax.ShapeDtypeStruct((N_PAD, D), jnp.float32),
            jax.ShapeDtypeStruct((N_PAD, 1), jnp.float32),
        ],
    )(deg3, x_p, W1)


def _tc_mid_body(p_ref, hp_ref, dis_ref, b_ref, w_ref, out_ref):
    z = (p_ref[0] + p_ref[1] + hp_ref[...]) * dis_ref[...] + b_ref[...]
    r = jnp.maximum(z, 0.0)
    out_ref[...] = jnp.dot(
        r, w_ref[...], preferred_element_type=jnp.float32
    ) * dis_ref[...]


def _tc_mid(parts, hp, dis2, b1, W2):
    return pl.pallas_call(
        _tc_mid_body,
        grid=(_GB,),
        in_specs=[
            pl.BlockSpec((NC, _RB, D), lambda i: (0, i, 0)),
            pl.BlockSpec((_RB, D), lambda i: (i, 0)),
            pl.BlockSpec((_RB, 1), lambda i: (i, 0)),
            pl.BlockSpec((1, D), lambda i: (0, 0)),
            pl.BlockSpec((D, D), lambda i: (0, 0)),
        ],
        out_specs=pl.BlockSpec((_RB, D), lambda i: (i, 0)),
        out_shape=jax.ShapeDtypeStruct((N_PAD, D), jnp.float32),
    )(parts, hp, dis2, b1, W2)


def _tc_final_body(p_ref, hp_ref, dis_ref, b_ref, out_ref):
    out_ref[...] = (
        p_ref[0] + p_ref[1] + hp_ref[...]
    ) * dis_ref[...] + b_ref[...]


def _tc_final(parts, hp, dis2, b2):
    return pl.pallas_call(
        _tc_final_body,
        grid=(_GB,),
        in_specs=[
            pl.BlockSpec((NC, _RB, D), lambda i: (0, i, 0)),
            pl.BlockSpec((_RB, D), lambda i: (i, 0)),
            pl.BlockSpec((_RB, 1), lambda i: (i, 0)),
            pl.BlockSpec((1, D), lambda i: (0, 0)),
        ],
        out_specs=pl.BlockSpec((_RB, D), lambda i: (i, 0)),
        out_shape=jax.ShapeDtypeStruct((N_PAD, D), jnp.float32),
    )(parts, hp, dis2, b2)


def kernel(x, edge_index, W1, b1, W2, b2):
    src = edge_index[0]
    dst = edge_index[1]
    pad_idx = jnp.arange(E_PAD - E, dtype=jnp.int32) % (N_PAD - N) + N
    src_p = jnp.concatenate([src, pad_idx]).reshape(NW, NBLK, K)
    dst_p = jnp.concatenate([dst, pad_idx]).reshape(NW, NBLK, K)
    x_p = jnp.pad(x, ((0, N_PAD - N), (0, 0)))

    degp = _deg_kernel(dst_p)
    hp1, dis2 = _tc1(degp[..., None], x_p, W1)
    parts1 = _agg_kernel(hp1, src_p, dst_p)
    hp2 = _tc_mid(parts1, hp1, dis2, b1.reshape(1, D), W2)
    parts2 = _agg_kernel(hp2, src_p, dst_p)
    out = _tc_final(parts2, hp2, dis2, b2.reshape(1, D))
    return out[:N]


# speedup vs baseline: 1.2526x; 1.2526x over previous
import functools

import jax
import jax.numpy as jnp
from jax import lax
from jax.experimental import pallas as pl
from jax.experimental.pallas import tpu as pltpu
from jax.experimental.pallas import tpu_sc as plsc

N = 10000
E = 320000
D = 128

NC = 2
NS = 16
NW = NC * NS
K = 128
NBLK = 80
E_PAD = NW * NBLK * K
N_PAD = 10240
R_SUB = N_PAD // NS
NCHUNK = 2
BPC = NBLK // NCHUNK

_mesh = plsc.VectorSubcoreMesh(
    core_axis_name="c", subcore_axis_name="s", num_cores=NC, num_subcores=NS
)


@functools.partial(
    pl.kernel,
    out_type=jax.ShapeDtypeStruct((NC, N_PAD), jnp.float32),
    mesh=_mesh,
    scratch_types=[
        pltpu.VMEM((NBLK, K), jnp.int32),
        pltpu.VMEM((R_SUB,), jnp.float32),
        pltpu.VMEM((K,), jnp.float32),
        pltpu.VMEM_SHARED((N_PAD,), jnp.float32),
    ],
)
def _deg_kernel(dst_hbm, out_hbm, dst_v, zero_v, ones_v, acc):
    cid = lax.axis_index("c")
    sid = lax.axis_index("s")
    wid = sid * NC + cid

    def fill(i, _):
        zero_v[pl.ds(i * 16, 16)] = jnp.zeros((16,), jnp.float32)
        return 0

    lax.fori_loop(0, R_SUB // 16, fill, 0)

    def fill1(i, _):
        ones_v[pl.ds(i * 16, 16)] = jnp.ones((16,), jnp.float32)
        return 0

    lax.fori_loop(0, K // 16, fill1, 0)

    pltpu.sync_copy(dst_hbm.at[wid], dst_v)
    pltpu.sync_copy(zero_v, acc.at[pl.ds(sid * R_SUB, R_SUB)])
    plsc.subcore_barrier()

    def body(j, _):
        pltpu.sync_copy(ones_v, acc.at[dst_v.at[j]], add=True)
        return 0

    lax.fori_loop(0, NBLK, body, 0)
    plsc.subcore_barrier()
    pltpu.sync_copy(
        acc.at[pl.ds(sid * R_SUB, R_SUB)],
        out_hbm.at[cid, pl.ds(sid * R_SUB, R_SUB)],
    )


def _make_agg(dtype):
  lanes = 16 if dtype == jnp.float32 else 32

  @functools.partial(
      pl.kernel,
      out_type=jax.ShapeDtypeStruct((NC, N_PAD, D), dtype),
      mesh=_mesh,
      scratch_types=[
          pltpu.VMEM((BPC, K), jnp.int32),
          pltpu.VMEM((BPC, K), jnp.int32),
          pltpu.VMEM((K, D), dtype),
          pltpu.VMEM((K, D), dtype),
          pltpu.VMEM_SHARED((N_PAD, D), dtype),
          pltpu.SemaphoreType.DMA,
          pltpu.SemaphoreType.DMA,
      ],
  )
  def _agg_kernel(hp_hbm, src_hbm, dst_hbm, out_hbm,
                  src_v, dst_v, buf0, buf1, acc, sem0, sem1):
    cid = lax.axis_index("c")
    sid = lax.axis_index("s")
    wid = sid * NC + cid

    if dtype == jnp.float32:
        def fill(i, _):
            buf0[i // 8, pl.ds((i % 8) * 16, 16)] = jnp.zeros((16,), dtype)
            return 0

        lax.fori_loop(0, K * 8, fill, 0)
    else:
        def fill(i, _):
            buf0[pl.ds((i // 8) * 2, 2), pl.ds((i % 8) * 16, 16)] = (
                jnp.zeros((2, 16), dtype)
            )
            return 0

        lax.fori_loop(0, (K // 2) * 8, fill, 0)
    for t in range(R_SUB // K):
        pltpu.sync_copy(buf0, acc.at[pl.ds(sid * R_SUB + t * K, K)])
    plsc.subcore_barrier()

    def chunk(c, _):
        pltpu.sync_copy(src_hbm.at[wid, pl.ds(c * BPC, BPC)], src_v)
        pltpu.sync_copy(dst_hbm.at[wid, pl.ds(c * BPC, BPC)], dst_v)
        pltpu.async_copy(hp_hbm.at[src_v.at[0]], buf0, sem0)
        pltpu.async_copy(hp_hbm.at[src_v.at[1]], buf1, sem1)

        def body(g, _):
            a = 2 * g
            pltpu.make_async_copy(hp_hbm.at[src_v.at[a]], buf0, sem0).wait()
            pltpu.sync_copy(buf0, acc.at[dst_v.at[a]], add=True)

            @pl.when(g < BPC // 2 - 1)
            def _():
                pltpu.async_copy(hp_hbm.at[src_v.at[a + 2]], buf0, sem0)

            pltpu.make_async_copy(hp_hbm.at[src_v.at[a + 1]], buf1, sem1).wait()
            pltpu.sync_copy(buf1, acc.at[dst_v.at[a + 1]], add=True)

            @pl.when(g < BPC // 2 - 1)
            def _():
                pltpu.async_copy(hp_hbm.at[src_v.at[a + 3]], buf1, sem1)

            return 0

        lax.fori_loop(0, BPC // 2, body, 0)
        return 0

    lax.fori_loop(0, NCHUNK, chunk, 0)
    plsc.subcore_barrier()
    pltpu.sync_copy(
        acc.at[pl.ds(sid * R_SUB, R_SUB)],
        out_hbm.at[cid, pl.ds(sid * R_SUB, R_SUB)],
    )

  return _agg_kernel


_agg_f32 = _make_agg(jnp.float32)


_GB = 10
_RB = N_PAD // _GB


def _tc_mm_body(x_ref, w_ref, out_ref):
    out_ref[...] = jnp.dot(
        x_ref[...], w_ref[...], preferred_element_type=jnp.float32
    )


def _tc_mm(x, W1):
    return pl.pallas_call(
        _tc_mm_body,
        grid=(_GB,),
        in_specs=[
            pl.BlockSpec((N // _GB, D), lambda i: (i, 0)),
            pl.BlockSpec((D, D), lambda i: (0, 0)),
        ],
        out_specs=pl.BlockSpec((N // _GB, D), lambda i: (i, 0)),
        out_shape=jax.ShapeDtypeStruct((N_PAD, D), jnp.float32),
    )(x, W1)


def _tc_scale_body(deg_ref, h_ref, hp_ref, dis_ref):
    deg = deg_ref[0] + deg_ref[1] + 1.0
    dis = lax.rsqrt(deg)
    dis_ref[...] = dis
    hp_ref[...] = h_ref[...] * dis


def _tc_scale(deg3, h1x):
    return pl.pallas_call(
        _tc_scale_body,
        grid=(_GB,),
        in_specs=[
            pl.BlockSpec((NC, _RB, 1), lambda i: (0, i, 0)),
            pl.BlockSpec((_RB, D), lambda i: (i, 0)),
        ],
        out_specs=[
            pl.BlockSpec((_RB, D), lambda i: (i, 0)),
            pl.BlockSpec((_RB, 1), lambda i: (i, 0)),
        ],
        out_shape=[
            jax.ShapeDtypeStruct((N_PAD, D), jnp.float32),
            jax.ShapeDtypeStruct((N_PAD, 1), jnp.float32),
        ],
    )(deg3, h1x)


def _tc_mid_body(p_ref, hp_ref, dis_ref, b_ref, w_ref, out_ref):
    z = (p_ref[0] + p_ref[1] + hp_ref[...]) * dis_ref[...] + b_ref[...]
    r = jnp.maximum(z, 0.0)
    out_ref[...] = jnp.dot(
        r, w_ref[...], preferred_element_type=jnp.float32
    ) * dis_ref[...]


def _tc_mid(parts, hp, dis2, b1, W2):
    return pl.pallas_call(
        _tc_mid_body,
        grid=(_GB,),
        in_specs=[
            pl.BlockSpec((NC, _RB, D), lambda i: (0, i, 0)),
            pl.BlockSpec((_RB, D), lambda i: (i, 0)),
            pl.BlockSpec((_RB, 1), lambda i: (i, 0)),
            pl.BlockSpec((1, D), lambda i: (0, 0)),
            pl.BlockSpec((D, D), lambda i: (0, 0)),
        ],
        out_specs=pl.BlockSpec((_RB, D), lambda i: (i, 0)),
        out_shape=jax.ShapeDtypeStruct((N_PAD, D), jnp.float32),
    )(parts, hp, dis2, b1, W2)


def _tc_final_body(p_ref, hp_ref, dis_ref, b_ref, out_ref):
    out_ref[...] = (
        p_ref[0] + p_ref[1] + hp_ref[...]
    ) * dis_ref[...] + b_ref[...]


_RBF = N // _GB


def _tc_final(parts, hp, dis2, b2):
    return pl.pallas_call(
        _tc_final_body,
        grid=(_GB,),
        in_specs=[
            pl.BlockSpec((NC, _RBF, D), lambda i: (0, i, 0)),
            pl.BlockSpec((_RBF, D), lambda i: (i, 0)),
            pl.BlockSpec((_RBF, 1), lambda i: (i, 0)),
            pl.BlockSpec((1, D), lambda i: (0, 0)),
        ],
        out_specs=pl.BlockSpec((_RBF, D), lambda i: (i, 0)),
        out_shape=jax.ShapeDtypeStruct((N, D), jnp.float32),
    )(parts, hp, dis2, b2)


def kernel(x, edge_index, W1, b1, W2, b2):
    src = edge_index[0]
    dst = edge_index[1]
    pad_idx = jnp.arange(E_PAD - E, dtype=jnp.int32) % (N_PAD - N) + N
    src_p = jnp.concatenate([src, pad_idx]).reshape(NW, NBLK, K)
    dst_p = jnp.concatenate([dst, pad_idx]).reshape(NW, NBLK, K)

    h1x = _tc_mm(x, W1)
    degp = _deg_kernel(dst_p)
    hp1, dis2 = _tc_scale(degp[..., None], h1x)
    parts1 = _agg_f32(hp1, src_p, dst_p)
    hp2 = _tc_mid(parts1, hp1, dis2, b1.reshape(1, D), W2)
    parts2 = _agg_f32(hp2, src_p, dst_p)
    return _tc_final(parts2, hp2, dis2, b2.reshape(1, D))
